# Initial kernel scaffold; baseline (speedup 1.0000x reference)
import jax
import jax.numpy as jnp
from jax.experimental import pallas as pl


def kernel(acquired_kspace, acquiring_kspace, params):
    raise NotImplementedError("write your pallas kernel here")


# speedup vs baseline: 1.5094x; 1.5094x over previous
import jax
import jax.numpy as jnp
from jax.experimental import pallas as pl
from jax.experimental.pallas import tpu as pltpu

_ANY = pltpu.MemorySpace.HBM
_EPS = 1e-5
_SLOPE = 0.2


def _leaky(x, slope):
    if slope == 1.0:
        return x
    return jnp.where(x >= 0, x, slope * x)


def _rsqrt(x):
    r = jax.lax.rsqrt(x)
    r = r * (1.5 - 0.5 * x * r * r)
    return r * (1.5 - 0.5 * x * r * r)


def _sqrt(x):
    xs = jnp.maximum(x, 1e-30)
    return x * _rsqrt(xs)


def _norm_params(stats_ref, n, c):
    s = stats_ref[0, 0:1, :]
    ss = stats_ref[0, 1:2, :]
    mean = s * (1.0 / n)
    var = ss * (1.0 / n) - mean * mean
    denom = _sqrt(var + _EPS)
    return mean.reshape(1, 1, c), denom.reshape(1, 1, c)


def _dma_slab(x_hbm, slab, sem, b, s, n_tiles, rows_main, halo):
    t0 = s * rows_main
    cp_main = pltpu.make_async_copy(
        x_hbm.at[b, pl.ds(t0, rows_main)], slab.at[pl.ds(halo, rows_main)], sem)
    cp_main.start()
    if n_tiles > 1:
        @pl.when(s > 0)
        def _():
            pltpu.make_async_copy(
                x_hbm.at[b, pl.ds(t0 - halo, halo)], slab.at[pl.ds(0, halo)],
                sem).start()

        @pl.when(s < n_tiles - 1)
        def _():
            pltpu.make_async_copy(
                x_hbm.at[b, pl.ds(t0 + rows_main, halo)],
                slab.at[pl.ds(halo + rows_main, halo)], sem).start()
        cp_main.wait()

        @pl.when(s > 0)
        def _():
            pltpu.make_async_copy(
                x_hbm.at[b, pl.ds(t0 - halo, halo)], slab.at[pl.ds(0, halo)],
                sem).wait()

        @pl.when(s < n_tiles - 1)
        def _():
            pltpu.make_async_copy(
                x_hbm.at[b, pl.ds(t0 + rows_main, halo)],
                slab.at[pl.ds(halo + rows_main, halo)], sem).wait()
    else:
        cp_main.wait()


def _row_guard_mask(x, s, n_tiles, rows, q0, th):
    r = jax.lax.broadcasted_iota(jnp.int32, (rows, 1, 1), 0) + q0
    keep = ((r > 0) | (s > 0)) & ((r < th + 1) | (s < n_tiles - 1))
    return jnp.where(keep, x, 0.0)


def _pool2(x):
    r, w, c = x.shape
    a = x.reshape(r // 2, 2, w // 2, 2, c)
    return (a[:, 0, :, 0] + a[:, 0, :, 1] + a[:, 1, :, 0] + a[:, 1, :, 1]) * 0.25


def _make_conv3x3(b_sz, h_in, w_in, cins, cout, *, pool, first, n_tiles):
    h_out = h_in // 2 if pool else h_in
    w_out = w_in // 2 if pool else w_in
    th = h_out // n_tiles
    rows_main = 2 * th if pool else th
    halo = 2 if pool else 1
    slab_rows = rows_main + 2 * halo
    n_inputs = len(cins)
    cin_total = 2 if first else sum(cins)
    m = th * w_out
    n_prod = h_in * w_in

    def body(*refs):
        xs = refs[:n_inputs]
        idx = n_inputs
        if first:
            stats = [None] * n_inputs
        else:
            stats = refs[idx:idx + n_inputs]
            idx += n_inputs
        w_ref = refs[idx]
        b_ref = refs[idx + 1]
        y_ref = refs[idx + 2]
        ystats_ref = refs[idx + 3]
        slabs = refs[idx + 4:idx + 4 + n_inputs]
        xp_ref = refs[idx + 4 + n_inputs]
        acc_ref = refs[idx + 5 + n_inputs]
        sems = refs[idx + 6 + n_inputs:idx + 6 + 2 * n_inputs]

        b = pl.program_id(0)
        s = pl.program_id(1)

        for i in range(n_inputs):
            _dma_slab(xs[i], slabs[i], sems[i], b, s, n_tiles, rows_main, halo)

        c0 = 0
        for i in range(n_inputs):
            ci = 2 if first else cins[i]
            if not first:
                mean, denom = _norm_params(stats[i], n_prod, ci)
            qc = 8
            for q0 in range(0, th + 2, qc):
                n_r = min(qc, th + 2 - q0)
                if pool:
                    xv = slabs[i][2 * q0:2 * (q0 + n_r)]
                    xc = _pool2(_leaky((xv - mean) / denom, _SLOPE))
                elif first:
                    xv = slabs[i][q0:q0 + n_r]
                    sq = (xv * xv).reshape(n_r * w_in, 4)
                    rr = jax.lax.broadcasted_iota(jnp.int32, (4, 2), 0)
                    cc = jax.lax.broadcasted_iota(jnp.int32, (4, 2), 1)
                    pair = (rr // 2 == cc).astype(jnp.float32)
                    ssum = jax.lax.dot_general(
                        sq, pair, (((1,), (0,)), ((), ())),
                        preferred_element_type=jnp.float32,
                        precision=jax.lax.Precision.HIGHEST)
                    xc = _sqrt(ssum).reshape(n_r, w_in, ci)
                else:
                    xv = slabs[i][q0:q0 + n_r]
                    xc = _leaky((xv - mean) / denom, _SLOPE)
                xc = _row_guard_mask(xc, s, n_tiles, n_r, q0, th)
                xp_ref[q0:q0 + n_r, 1:w_out + 1, :] = xc
            xp_ref[:, 0:1, :] = jnp.zeros((th + 2, 1, ci), jnp.float32)
            xp_ref[:, w_out + 1:w_out + 2, :] = jnp.zeros(
                (th + 2, 1, ci), jnp.float32)
            cp = max(-(-ci // 128), -(-cout // 128)) * 128
            rchunk = th
            while (rchunk * w_out * cp * 4 > 1_300_000 and rchunk % 2 == 0
                   and rchunk > 8):
                rchunk //= 2
            for r0 in range(0, th, rchunk):
                mc = rchunk * w_out
                part = jnp.zeros((mc, cout), jnp.float32)
                for k in range(9):
                    dy, dx = k // 3, k % 3
                    lhs = xp_ref[dy + r0:dy + r0 + rchunk,
                                 dx:dx + w_out, :].reshape(mc, ci)
                    part = part + jax.lax.dot_general(
                        lhs, w_ref[k, c0:c0 + ci, :],
                        (((1,), (0,)), ((), ())),
                        preferred_element_type=jnp.float32,
                        precision=jax.lax.Precision.DEFAULT)
                if i == 0:
                    acc_ref[r0 * w_out:r0 * w_out + mc, :] = part
                else:
                    acc_ref[r0 * w_out:r0 * w_out + mc, :] += part
            c0 += ci

        acc = acc_ref[...] + b_ref[...]
        ysum = jnp.sum(acc, axis=0, keepdims=True)
        ysq = jnp.sum(acc * acc, axis=0, keepdims=True)

        @pl.when(s == 0)
        def _():
            ystats_ref[0, 0:1, :] = ysum
            ystats_ref[0, 1:2, :] = ysq

        @pl.when(s > 0)
        def _():
            ystats_ref[0, 0:1, :] += ysum
            ystats_ref[0, 1:2, :] += ysq

        y_ref[0] = acc.reshape(th, w_out, cout)

    in_specs = [pl.BlockSpec(memory_space=_ANY)] * n_inputs
    if not first:
        in_specs += [pl.BlockSpec((1, 2, ci), lambda b, s: (b, 0, 0))
                     for ci in cins]
    in_specs += [
        pl.BlockSpec((9, cin_total, cout), lambda b, s: (0, 0, 0)),
        pl.BlockSpec((1, cout), lambda b, s: (0, 0)),
    ]
    out_specs = [
        pl.BlockSpec((1, th, w_out, cout), lambda b, s: (b, s, 0, 0)),
        pl.BlockSpec((1, 2, cout), lambda b, s: (b, 0, 0)),
    ]
    ci_conv = 2 if first else cins[0]
    scratch = ([pltpu.VMEM((slab_rows, w_in, ci), jnp.float32) for ci in cins]
               + [pltpu.VMEM((th + 2, w_out + 2, ci_conv), jnp.float32),
                  pltpu.VMEM((m, cout), jnp.float32)]
               + [pltpu.SemaphoreType.DMA] * n_inputs)
    return pl.pallas_call(
        body,
        grid=(b_sz, n_tiles),
        in_specs=in_specs,
        out_specs=out_specs,
        out_shape=[
            jax.ShapeDtypeStruct((b_sz, h_out, w_out, cout), jnp.float32),
            jax.ShapeDtypeStruct((b_sz, 2, cout), jnp.float32),
        ],
        scratch_shapes=scratch,
        compiler_params=pltpu.CompilerParams(
            dimension_semantics=("arbitrary", "arbitrary")),
    )


def _make_convt(b_sz, h_in, w_in, cin, cout, *, n_tiles):
    th = h_in // n_tiles
    m = th * w_in
    n_prod = h_in * w_in

    def body(x_hbm, stats_ref, w_ref, y_ref, ystats_ref, slab, sem):
        b = pl.program_id(0)
        s = pl.program_id(1)
        pltpu.make_async_copy(
            x_hbm.at[b, pl.ds(s * th, th)], slab, sem).start()
        pltpu.make_async_copy(
            x_hbm.at[b, pl.ds(s * th, th)], slab, sem).wait()
        mean, denom = _norm_params(stats_ref, n_prod, cin)
        ysum = jnp.zeros((1, cout), jnp.float32)
        ysq = jnp.zeros((1, cout), jnp.float32)
        cp = max(-(-cin // 128), -(-cout // 128)) * 128
        rc = th
        while rc * w_in * cp * 4 > 1_300_000 and rc % 2 == 0 and rc > 8:
            rc //= 2
        for r0 in range(0, th, rc):
            xc = _leaky((slab[r0:r0 + rc] - mean) / denom, _SLOPE)
            lhs = xc.reshape(rc * w_in, cin)
            rows = []
            for di in (0, 1):
                cols = []
                for dj in (0, 1):
                    y = jax.lax.dot_general(
                        lhs, w_ref[2 * di + dj],
                        (((1,), (0,)), ((), ())),
                        preferred_element_type=jnp.float32,
                        precision=jax.lax.Precision.DEFAULT)
                    ysum = ysum + jnp.sum(y, axis=0, keepdims=True)
                    ysq = ysq + jnp.sum(y * y, axis=0, keepdims=True)
                    cols.append(y.reshape(rc, w_in, 1, cout))
                rows.append(jnp.concatenate(cols, axis=2).reshape(
                    rc, 1, 2 * w_in, cout))
            y_ref[0, 2 * r0:2 * (r0 + rc)] = jnp.concatenate(
                rows, axis=1).reshape(2 * rc, 2 * w_in, cout)

        @pl.when(s == 0)
        def _():
            ystats_ref[0, 0:1, :] = ysum
            ystats_ref[0, 1:2, :] = ysq

        @pl.when(s > 0)
        def _():
            ystats_ref[0, 0:1, :] += ysum
            ystats_ref[0, 1:2, :] += ysq

    return pl.pallas_call(
        body,
        grid=(b_sz, n_tiles),
        in_specs=[
            pl.BlockSpec(memory_space=_ANY),
            pl.BlockSpec((1, 2, cin), lambda b, s: (b, 0, 0)),
            pl.BlockSpec((4, cin, cout), lambda b, s: (0, 0, 0)),
        ],
        out_specs=[
            pl.BlockSpec((1, 2 * th, 2 * w_in, cout),
                         lambda b, s: (b, s, 0, 0)),
            pl.BlockSpec((1, 2, cout), lambda b, s: (b, 0, 0)),
        ],
        out_shape=[
            jax.ShapeDtypeStruct((b_sz, 2 * h_in, 2 * w_in, cout),
                                 jnp.float32),
            jax.ShapeDtypeStruct((b_sz, 2, cout), jnp.float32),
        ],
        scratch_shapes=[pltpu.VMEM((th, w_in, cin), jnp.float32),
                        pltpu.SemaphoreType.DMA],
        compiler_params=pltpu.CompilerParams(
            dimension_semantics=("arbitrary", "arbitrary")),
    )


def _make_final(b_sz, h, w, cin, *, n_tiles):
    th = h // n_tiles
    m = th * w
    n_prod = h * w

    def body(x_hbm, stats_ref, w_ref, b_ref, y_ref, slab, sem):
        b = pl.program_id(0)
        s = pl.program_id(1)
        pltpu.make_async_copy(
            x_hbm.at[b, pl.ds(s * th, th)], slab, sem).start()
        pltpu.make_async_copy(
            x_hbm.at[b, pl.ds(s * th, th)], slab, sem).wait()
        mean, denom = _norm_params(stats_ref, n_prod, cin)
        for r0 in range(0, th, 8):
            xc = _leaky((slab[r0:r0 + 8] - mean) / denom, _SLOPE)
            y = jax.lax.dot_general(
                xc.reshape(8 * w, cin), w_ref[...],
                (((1,), (0,)), ((), ())),
                preferred_element_type=jnp.float32,
                precision=jax.lax.Precision.DEFAULT) + b_ref[...]
            y_ref[0, r0:r0 + 8] = y.reshape(8, w, 1)

    return pl.pallas_call(
        body,
        grid=(b_sz, n_tiles),
        in_specs=[
            pl.BlockSpec(memory_space=_ANY),
            pl.BlockSpec((1, 2, cin), lambda b, s: (b, 0, 0)),
            pl.BlockSpec((cin, 1), lambda b, s: (0, 0)),
            pl.BlockSpec((1, 1), lambda b, s: (0, 0)),
        ],
        out_specs=[
            pl.BlockSpec((1, th, w, 1), lambda b, s: (b, s, 0, 0)),
        ],
        out_shape=[
            jax.ShapeDtypeStruct((b_sz, h, w, 1), jnp.float32),
        ],
        scratch_shapes=[pltpu.VMEM((th, w, cin), jnp.float32),
                        pltpu.SemaphoreType.DMA],
        compiler_params=pltpu.CompilerParams(
            dimension_semantics=("arbitrary", "arbitrary")),
    )


def _conv_w(p):
    return jnp.transpose(p["w"], (2, 3, 1, 0)).reshape(
        9, p["w"].shape[1], p["w"].shape[0])


def _convt_w(w):
    wt = jnp.transpose(w, (1, 2, 3, 0))
    return jnp.stack([wt[:, 1, 1], wt[:, 1, 0], wt[:, 0, 1], wt[:, 0, 0]], 0)


@jax.jit
def kernel(acquired_kspace, acquiring_kspace, params):
    b_sz, _, h, _, _ = acquired_kspace.shape
    chans = params["down0"]["c1"]["w"].shape[0]

    ri = jnp.concatenate(
        [acquired_kspace[:, 0], acquiring_kspace[:, 0]], axis=-1)

    def conv(xs, stats, p, h_in, cins, cout, *, pool=False, first=False,
             n_tiles=1):
        w9 = _conv_w(p)
        bias = p["b"].reshape(1, cout)
        fn = _make_conv3x3(b_sz, h_in, h_in, cins, cout, pool=pool,
                           first=first, n_tiles=n_tiles)
        args = list(xs) + ([] if first else list(stats)) + [w9, bias]
        return fn(*args)

    conv_tiles = {384: 12, 192: 4, 96: 1, 48: 1, 24: 1}
    pool_tiles = {192: 8, 96: 4, 48: 1, 24: 1}

    c = chans
    y0, s0 = conv([ri], None, params["down0"]["c1"], h, [4], c, first=True,
                  n_tiles=conv_tiles[h])
    y1, s1 = conv([y0], [s0], params["down0"]["c2"], h, [c], c,
                  n_tiles=conv_tiles[h])
    skips = [(y1, s1, c, h)]
    x_prev, s_prev, c_prev, h_prev = y1, s1, c, h
    for l in range(1, 4):
        c_out = c_prev * 2
        h_out = h_prev // 2
        a0, t0 = conv([x_prev], [s_prev], params["down%d" % l]["c1"], h_prev,
                      [c_prev], c_out, pool=True, n_tiles=pool_tiles[h_out])
        a1, t1 = conv([a0], [t0], params["down%d" % l]["c2"], h_out,
                      [c_out], c_out, n_tiles=conv_tiles[h_out])
        skips.append((a1, t1, c_out, h_out))
        x_prev, s_prev, c_prev, h_prev = a1, t1, c_out, h_out

    c_out = c_prev * 2
    h_out = h_prev // 2
    a0, t0 = conv([x_prev], [s_prev], params["bottleneck"]["c1"], h_prev,
                  [c_prev], c_out, pool=True, n_tiles=1)
    a1, t1 = conv([a0], [t0], params["bottleneck"]["c2"], h_out,
                  [c_out], c_out, n_tiles=1)
    x_prev, s_prev, c_prev, h_prev = a1, t1, c_out, h_out

    upt_tiles = {24: 1, 48: 1, 96: 2, 192: 8}
    for l in range(4):
        c_out = c_prev // 2
        wt = _convt_w(params["upt%d" % l]["w"])
        up_fn = _make_convt(b_sz, h_prev, h_prev, c_prev, c_out,
                            n_tiles=upt_tiles[h_prev])
        u, su = up_fn(x_prev, s_prev, wt)
        h_up = h_prev * 2
        skip, s_skip, c_skip, h_skip = skips.pop()
        assert h_skip == h_up and c_skip == c_out
        v0, sv0 = conv([u, skip], [su, s_skip], params["upc%d" % l]["c1"],
                       h_up, [c_out, c_skip], c_out, n_tiles=conv_tiles[h_up])
        v1, sv1 = conv([v0], [sv0], params["upc%d" % l]["c2"], h_up,
                       [c_out], c_out, n_tiles=conv_tiles[h_up])
        x_prev, s_prev, c_prev, h_prev = v1, sv1, c_out, h_up

    fw = params["final"]["w"].reshape(1, c_prev).T
    fb = params["final"]["b"].reshape(1, 1)
    fin = _make_final(b_sz, h_prev, h_prev, c_prev, n_tiles=8)
    out = fin(x_prev, s_prev, fw, fb)[0]
    return out[..., 0]
